# baseline (speedup 1.0000x reference)
import jax
import jax.numpy as jnp
from jax.experimental import pallas as pl

_BN_S = (1.0 + 1e-5) ** -0.5
_P = 1024
_K = 4


def _dense_bn(a, w, b):
    return jnp.maximum(
        jnp.dot(a.astype(w.dtype), w, preferred_element_type=jnp.float32)
        + b, 0.0) * _BN_S


def _edge_conv(x, rows, cols, wa_t, wb_t, b1, tail):
    f32 = jnp.float32
    g = jax.lax.dot_general(x, x, (((1,), (1,)), ((), ())),
                            preferred_element_type=f32)
    d2c = jnp.sum(x * x, axis=1, keepdims=True)
    d2r = jnp.max(jnp.where(rows == cols, g, -jnp.inf), axis=0, keepdims=True)
    d = jnp.maximum(d2c + d2r - 2.0 * g, 1.17549435e-38)
    keys = (jax.lax.bitcast_convert_type(d, jnp.int32) & ~1023) | cols
    keys = jax.lax.bitcast_convert_type(keys, f32)

    base = jnp.dot(x, wa_t, preferred_element_type=f32) + b1
    xg = x.astype(jnp.bfloat16)
    acc = None
    m = None
    for _ in range(_K):
        masked = keys if m is None else jnp.where(keys > m, keys, jnp.inf)
        m = jnp.min(masked, axis=1, keepdims=True)
        eq = keys == m
        oh = jnp.where(eq, 1.0, 0.0).astype(jnp.bfloat16)
        xj = jnp.dot(oh, xg, preferred_element_type=f32)
        h = jnp.maximum(
            base + jnp.dot((xj - x).astype(wb_t.dtype), wb_t,
                           preferred_element_type=f32),
            0.0) * _BN_S
        for (w_t, b) in tail:
            h = _dense_bn(h, w_t, b)
        acc = h if acc is None else jnp.maximum(acc, h)
    return acc


def _stn_points_body(pos_ref, sW1t, sb1, sW2t, sb2, sW3t, sb3, out_ref):
    t = _dense_bn(pos_ref[0], sW1t[...], sb1[...])
    t = _dense_bn(t, sW2t[...], sb2[...])
    t = _dense_bn(t, sW3t[...], sb3[...])
    out_ref[0] = jnp.max(t, axis=0, keepdims=True)


def _stn_fc_body(g_ref, sF1t, sfb1, sF2t, sfb2, sF3t, sfb3e, out_ref):
    g = _dense_bn(g_ref[...], sF1t[...], sfb1[...])
    g = _dense_bn(g, sF2t[...], sfb2[...])
    out_ref[...] = (jnp.dot(g, sF3t[...], preferred_element_type=jnp.float32)
                    + sfb3e[...])


def _convs_body(pos_ref, t9_ref,
                c1At, c1Bt, c1b1, c1W2t, c1b2, c1W3t, c1b3,
                c2At, c2Bt, c2b1,
                lAt, lBt, lb,
                out_ref):
    f32 = jnp.float32
    pos = pos_ref[0]
    t9 = t9_ref[0]

    tmat = jnp.concatenate([t9[:, 0:3], t9[:, 3:6], t9[:, 6:9]], axis=0)
    x = jnp.dot(pos, tmat, preferred_element_type=f32)

    rows = jax.lax.broadcasted_iota(jnp.int32, (_P, _P), 0)
    cols = jax.lax.broadcasted_iota(jnp.int32, (_P, _P), 1)

    x1 = _edge_conv(x, rows, cols, c1At[...], c1Bt[...], c1b1[...],
                    [(c1W2t[...], c1b2[...]), (c1W3t[...], c1b3[...])])

    x2 = _edge_conv(x1, rows, cols, c2At[...], c2Bt[...], c2b1[...], [])

    h = jnp.maximum(
        jnp.dot(x1.astype(jnp.bfloat16), lAt[...],
                preferred_element_type=f32)
        + jnp.dot(x2.astype(jnp.bfloat16), lBt[...],
                  preferred_element_type=f32) + lb[...],
        0.0) * _BN_S
    out_ref[0] = jnp.max(h, axis=0, keepdims=True)


def _head_body(gg_ref, mW1t, mb1, mW2t, mb2, mW3t, mb3, out_ref):
    f32 = jnp.float32
    m = _dense_bn(gg_ref[...], mW1t[...], mb1[...])
    m = _dense_bn(m, mW2t[...], mb2[...])
    logits = jnp.dot(m, mW3t[...], preferred_element_type=f32) + mb3[...]
    z = logits - jnp.max(logits, axis=1, keepdims=True)
    out_ref[...] = z - jnp.log(jnp.sum(jnp.exp(z), axis=1, keepdims=True))


def _full_spec(w):
    return pl.BlockSpec(w.shape, lambda *_, n=w.ndim: (0,) * n)


def _single_call(body, args, out_shape):
    return pl.pallas_call(
        body,
        in_specs=[_full_spec(a) for a in args],
        out_specs=_full_spec(jax.ShapeDtypeStruct(out_shape, jnp.float32)),
        out_shape=jax.ShapeDtypeStruct(out_shape, jnp.float32),
    )(*args)


def kernel(pos, batch, sW1, sb1, sW2, sb2, sW3, sb3, sF1, sfb1, sF2, sfb2, sF3, sfb3, c1W1, c1b1, c1W2, c1b2, c1W3, c1b3, c2W1, c2b1, lW, lb, mW1, mb1, mW2, mb2, mW3, mb3):
    f32 = jnp.float32
    nb = pos.shape[0] // _P
    pos3 = pos.astype(f32).reshape(nb, _P, 3)
    eye9 = jnp.eye(3, dtype=f32).reshape(9)

    stn_w = (sW1.T, sb1[None], sW2.T, sb2[None], sW3.T, sb3[None])
    gmax = pl.pallas_call(
        _stn_points_body,
        grid=(nb,),
        in_specs=[pl.BlockSpec((1, _P, 3), lambda b: (b, 0, 0))]
        + [_full_spec(w) for w in stn_w],
        out_specs=pl.BlockSpec((1, 1, 1024), lambda b: (b, 0, 0)),
        out_shape=jax.ShapeDtypeStruct((nb, 1, 1024), f32),
    )(pos3, *stn_w)

    t9 = _single_call(
        _stn_fc_body,
        (gmax.reshape(nb, 1024), sF1.T, sfb1[None], sF2.T, sfb2[None],
         sF3.T, (sfb3 + eye9)[None]),
        (nb, 9))

    conv_w = (
        c1W1[:, :3].T, c1W1[:, 3:].T, c1b1[None], c1W2.T, c1b2[None],
        c1W3.T, c1b3[None],
        c2W1[:, :64].T, c2W1[:, 64:].T.astype(jnp.bfloat16), c2b1[None],
        lW[:, :64].T.astype(jnp.bfloat16),
        lW[:, 64:].T.astype(jnp.bfloat16), lb[None],
    )
    gg = pl.pallas_call(
        _convs_body,
        grid=(nb,),
        in_specs=[pl.BlockSpec((1, _P, 3), lambda b: (b, 0, 0)),
                  pl.BlockSpec((1, 1, 9), lambda b: (b, 0, 0))]
        + [_full_spec(w) for w in conv_w],
        out_specs=pl.BlockSpec((1, 1, 1024), lambda b: (b, 0, 0)),
        out_shape=jax.ShapeDtypeStruct((nb, 1, 1024), f32),
    )(pos3, t9.reshape(nb, 1, 9), *conv_w)

    return _single_call(
        _head_body,
        (gg.reshape(nb, 1024), mW1.T, mb1[None], mW2.T, mb2[None],
         mW3.T, mb3[None]),
        (nb, 40))


# speedup vs baseline: 1.0384x; 1.0384x over previous
import jax
import jax.numpy as jnp
from jax.experimental import pallas as pl

_BN_S = (1.0 + 1e-5) ** -0.5
_P = 1024
_K = 4


def _dense_bn(a, w, b):
    return jnp.maximum(
        jnp.dot(a.astype(w.dtype), w, preferred_element_type=jnp.float32)
        + b, 0.0) * _BN_S


def _edge_conv(x, rows, cols, wa_t, wb_t, b1, tail,
               gather_dtype=jnp.float32):
    f32 = jnp.float32
    g = jax.lax.dot_general(x, x, (((1,), (1,)), ((), ())),
                            preferred_element_type=f32)
    d2c = jnp.sum(x * x, axis=1, keepdims=True)
    d2r = jnp.max(jnp.where(rows == cols, g, -jnp.inf), axis=0, keepdims=True)
    d = jnp.maximum(d2c + d2r - 2.0 * g, 1.17549435e-38)
    keys = (jax.lax.bitcast_convert_type(d, jnp.int32) & ~1023) | cols
    keys = jax.lax.bitcast_convert_type(keys, f32)

    base = jnp.dot(x, wa_t, preferred_element_type=f32) + b1
    xg = x.astype(gather_dtype)
    acc = None
    for _ in range(_K):
        m = jnp.min(keys, axis=1, keepdims=True)
        eq = keys == m
        oh = jnp.where(eq, 1.0, 0.0).astype(gather_dtype)
        keys = jnp.where(eq, jnp.inf, keys)
        xj = jnp.dot(oh, xg, preferred_element_type=f32)
        h = jnp.maximum(
            base + jnp.dot((xj - x).astype(wb_t.dtype), wb_t,
                           preferred_element_type=f32),
            0.0) * _BN_S
        for (w_t, b) in tail:
            h = _dense_bn(h, w_t, b)
        acc = h if acc is None else jnp.maximum(acc, h)
    return acc


def _stn_points_body(pos_ref, sW1t, sb1, sW2t, sb2, sW3t, sb3, out_ref):
    t = _dense_bn(pos_ref[0], sW1t[...], sb1[...])
    t = _dense_bn(t, sW2t[...], sb2[...])
    t = _dense_bn(t, sW3t[...], sb3[...])
    out_ref[0] = jnp.max(t, axis=0, keepdims=True)


def _stn_fc_body(g_ref, sF1t, sfb1, sF2t, sfb2, sF3t, sfb3e, out_ref):
    g = _dense_bn(g_ref[...], sF1t[...], sfb1[...])
    g = _dense_bn(g, sF2t[...], sfb2[...])
    out_ref[...] = (jnp.dot(g, sF3t[...], preferred_element_type=jnp.float32)
                    + sfb3e[...])


def _convs_body(pos_ref, t9_ref,
                c1At, c1Bt, c1b1, c1W2t, c1b2, c1W3t, c1b3,
                c2At, c2Bt, c2b1,
                lAt, lBt, lb,
                out_ref):
    f32 = jnp.float32
    pos = pos_ref[0]
    t9 = t9_ref[0]

    tmat = jnp.concatenate([t9[:, 0:3], t9[:, 3:6], t9[:, 6:9]], axis=0)
    x = jnp.dot(pos, tmat, preferred_element_type=f32)

    rows = jax.lax.broadcasted_iota(jnp.int32, (_P, _P), 0)
    cols = jax.lax.broadcasted_iota(jnp.int32, (_P, _P), 1)

    x1 = _edge_conv(x, rows, cols, c1At[...], c1Bt[...], c1b1[...],
                    [(c1W2t[...], c1b2[...]), (c1W3t[...], c1b3[...])])

    x2 = _edge_conv(x1, rows, cols, c2At[...], c2Bt[...], c2b1[...], [],
                    gather_dtype=jnp.bfloat16)

    h = jnp.maximum(
        jnp.dot(x1.astype(jnp.bfloat16), lAt[...],
                preferred_element_type=f32)
        + jnp.dot(x2.astype(jnp.bfloat16), lBt[...],
                  preferred_element_type=f32) + lb[...],
        0.0) * _BN_S
    out_ref[0] = jnp.max(h, axis=0, keepdims=True)


def _head_body(gg_ref, mW1t, mb1, mW2t, mb2, mW3t, mb3, out_ref):
    f32 = jnp.float32
    m = _dense_bn(gg_ref[...], mW1t[...], mb1[...])
    m = _dense_bn(m, mW2t[...], mb2[...])
    logits = jnp.dot(m, mW3t[...], preferred_element_type=f32) + mb3[...]
    z = logits - jnp.max(logits, axis=1, keepdims=True)
    out_ref[...] = z - jnp.log(jnp.sum(jnp.exp(z), axis=1, keepdims=True))


def _full_spec(w):
    return pl.BlockSpec(w.shape, lambda *_, n=w.ndim: (0,) * n)


def _single_call(body, args, out_shape):
    return pl.pallas_call(
        body,
        in_specs=[_full_spec(a) for a in args],
        out_specs=_full_spec(jax.ShapeDtypeStruct(out_shape, jnp.float32)),
        out_shape=jax.ShapeDtypeStruct(out_shape, jnp.float32),
    )(*args)


def kernel(pos, batch, sW1, sb1, sW2, sb2, sW3, sb3, sF1, sfb1, sF2, sfb2, sF3, sfb3, c1W1, c1b1, c1W2, c1b2, c1W3, c1b3, c2W1, c2b1, lW, lb, mW1, mb1, mW2, mb2, mW3, mb3):
    f32 = jnp.float32
    nb = pos.shape[0] // _P
    pos3 = pos.astype(f32).reshape(nb, _P, 3)
    eye9 = jnp.eye(3, dtype=f32).reshape(9)

    stn_w = (sW1.T, sb1[None], sW2.T, sb2[None], sW3.T, sb3[None])
    gmax = pl.pallas_call(
        _stn_points_body,
        grid=(nb,),
        in_specs=[pl.BlockSpec((1, _P, 3), lambda b: (b, 0, 0))]
        + [_full_spec(w) for w in stn_w],
        out_specs=pl.BlockSpec((1, 1, 1024), lambda b: (b, 0, 0)),
        out_shape=jax.ShapeDtypeStruct((nb, 1, 1024), f32),
    )(pos3, *stn_w)

    t9 = _single_call(
        _stn_fc_body,
        (gmax.reshape(nb, 1024), sF1.T, sfb1[None], sF2.T, sfb2[None],
         sF3.T, (sfb3 + eye9)[None]),
        (nb, 9))

    conv_w = (
        c1W1[:, :3].T, c1W1[:, 3:].T, c1b1[None], c1W2.T, c1b2[None],
        c1W3.T, c1b3[None],
        c2W1[:, :64].T, c2W1[:, 64:].T.astype(jnp.bfloat16), c2b1[None],
        lW[:, :64].T.astype(jnp.bfloat16),
        lW[:, 64:].T.astype(jnp.bfloat16), lb[None],
    )
    gg = pl.pallas_call(
        _convs_body,
        grid=(nb,),
        in_specs=[pl.BlockSpec((1, _P, 3), lambda b: (b, 0, 0)),
                  pl.BlockSpec((1, 1, 9), lambda b: (b, 0, 0))]
        + [_full_spec(w) for w in conv_w],
        out_specs=pl.BlockSpec((1, 1, 1024), lambda b: (b, 0, 0)),
        out_shape=jax.ShapeDtypeStruct((nb, 1, 1024), f32),
    )(pos3, t9.reshape(nb, 1, 9), *conv_w)

    return _single_call(
        _head_body,
        (gg.reshape(nb, 1024), mW1.T, mb1[None], mW2.T, mb2[None],
         mW3.T, mb3[None]),
        (nb, 40))
